# baseline (speedup 1.0000x reference)
import dataclasses
import functools

import jax
import jax.numpy as jnp
from jax import lax
from jax.experimental import pallas as pl
from jax.experimental.pallas import tpu as pltpu
from jax.experimental.pallas import tpu_sc as plsc

D = 128
NT = 2
NEG = 0.2
NC = 2
NS = 16
NW = NC * NS
LANES = 16
CHUNK = 128
IB = 16
HD = D // NC
HB = HD // 2
BN = 1024


def _ceil_to(v, m):
    return -(-v // m) * m


def _sc_params():
    cp = pltpu.CompilerParams()
    if "needs_layout_passes" in pltpu.CompilerParams.__dataclass_fields__:
        cp = dataclasses.replace(cp, needs_layout_passes=False)
    if "use_tc_tiling_on_sc" in pltpu.CompilerParams.__dataclass_fields__:
        cp = dataclasses.replace(cp, use_tc_tiling_on_sc=False)
    return cp


def _precompute(x_pad, weight, att, root_w, root_b2, n_pad):
    grid = (n_pad // BN,)

    def body(x_ref, w_ref, a_ref, rw_ref, rb_ref, xw_ref, sd_ref, root_ref):
        xb = x_ref[...]
        w = w_ref[...]
        xw0 = jnp.dot(xb, w[0], preferred_element_type=jnp.float32)
        xw1 = jnp.dot(xb, w[1], preferred_element_type=jnp.float32)
        a = a_ref[...]
        s0 = jnp.sum(xw0 * a[0, :D][None, :], axis=1)
        s1 = jnp.sum(xw1 * a[1, :D][None, :], axis=1)
        d0 = jnp.sum(xw0 * a[0, D:][None, :], axis=1)
        d1 = jnp.sum(xw1 * a[1, D:][None, :], axis=1)
        sd_ref[...] = jnp.stack([s0, s1, d0, d1, s0, s1, d0, d1], axis=0)
        lo = jnp.stack([xw0[:, :HD], xw1[:, :HD]])
        hi = jnp.stack([xw0[:, HD:], xw1[:, HD:]])
        xw_ref[...] = jnp.stack([lo, hi])
        root_ref[...] = (jnp.dot(xb, rw_ref[...],
                                 preferred_element_type=jnp.float32)
                         + rb_ref[...])

    return pl.pallas_call(
        body,
        grid=grid,
        in_specs=[
            pl.BlockSpec((BN, D), lambda i: (i, 0)),
            pl.BlockSpec((NT, D, D), lambda i: (0, 0, 0)),
            pl.BlockSpec((NT, 2 * D), lambda i: (0, 0)),
            pl.BlockSpec((D, D), lambda i: (0, 0)),
            pl.BlockSpec((1, D), lambda i: (0, 0)),
        ],
        out_specs=[
            pl.BlockSpec((NC, NT, BN, HD), lambda i: (0, 0, i, 0)),
            pl.BlockSpec((8, BN), lambda i: (0, i)),
            pl.BlockSpec((BN, D), lambda i: (i, 0)),
        ],
        out_shape=[
            jax.ShapeDtypeStruct((NC, NT, n_pad, HD), jnp.float32),
            jax.ShapeDtypeStruct((8, n_pad), jnp.float32),
            jax.ShapeDtypeStruct((n_pad, D), jnp.float32),
        ],
    )(x_pad, weight, att, root_w, root_b2)


def _sc_phase1(s_flat, d_flat, cmb, n_pad, nblk):
    mesh = plsc.VectorSubcoreMesh(core_axis_name="c", subcore_axis_name="s")

    @functools.partial(
        pl.kernel,
        compiler_params=_sc_params(),
        out_type=[
            jax.ShapeDtypeStruct((NS, nblk, 3, IB, CHUNK), jnp.int32),
            jax.ShapeDtypeStruct((NW, n_pad), jnp.float32),
        ],
        mesh=mesh,
        scratch_types=[
            pltpu.VMEM((3, IB, CHUNK), jnp.int32),
            pltpu.VMEM((3, IB, CHUNK), jnp.int32),
            pltpu.VMEM((NT * n_pad,), jnp.float32),
            pltpu.VMEM((NT * n_pad,), jnp.float32),
            pltpu.VMEM((n_pad,), jnp.float32),
        ],
    )
    def k(s_hbm, d_hbm, cmb_hbm, pfx_hbm, den_hbm,
          cin_v, cout_v, s_v, d_v, den_v):
        cid = lax.axis_index("c")
        sid = lax.axis_index("s")

        zero16 = jnp.zeros((LANES,), jnp.float32)

        @pl.loop(0, n_pad, step=LANES)
        def _(i):
            den_v[pl.ds(i, LANES)] = zero16

        pltpu.sync_copy(s_hbm, s_v)
        pltpu.sync_copy(d_hbm, d_v)

        @pl.loop(cid, nblk, step=2)
        def _(blk):
            pltpu.sync_copy(cmb_hbm.at[sid, blk], cin_v)

            @pl.loop(0, IB)
            def _(ci):
                @plsc.parallel_loop(0, CHUNK, step=LANES, unroll=2)
                def _(j):
                    src16 = cin_v[0, ci, pl.ds(j, LANES)]
                    dst16 = cin_v[1, ci, pl.ds(j, LANES)]
                    typ16 = cin_v[2, ci, pl.ds(j, LANES)]
                    fs = typ16 * n_pad + src16
                    fd = typ16 * n_pad + dst16
                    sg = plsc.load_gather(s_v, [fs])
                    dg = plsc.load_gather(d_v, [fd])
                    logit = sg + dg
                    e = jnp.where(logit >= 0, logit, logit * NEG)
                    pe = jnp.exp(e)
                    cout_v[0, ci, pl.ds(j, LANES)] = fs
                    cout_v[1, ci, pl.ds(j, LANES)] = dst16
                    cout_v[2, ci, pl.ds(j, LANES)] = plsc.bitcast(pe,
                                                                  jnp.int32)
                    plsc.addupdate_scatter(den_v, [dst16], pe)

            pltpu.sync_copy(cout_v, pfx_hbm.at[sid, blk])

        pltpu.sync_copy(den_v, den_hbm.at[cid * NS + sid])

    return k(s_flat, d_flat, cmb)


def _sc_aggregate(xw2, pfx, n_pad, nblk):
    mesh = plsc.VectorSubcoreMesh(core_axis_name="c", subcore_axis_name="s")
    rows_per_tile = n_pad // NS
    nzero = rows_per_tile // CHUNK
    trows = NT * n_pad // NS
    nchunk = nblk * IB

    @functools.partial(
        pl.kernel,
        compiler_params=_sc_params(),
        out_type=jax.ShapeDtypeStruct((NC, n_pad, HD), jnp.float32),
        mesh=mesh,
        scratch_types=[
            pltpu.VMEM((2, 3, IB, CHUNK), jnp.int32),
            pltpu.VMEM((2, CHUNK, HB), jnp.int32),
            pltpu.VMEM((2, CHUNK, HD), jnp.float32),
            pltpu.VMEM_SHARED((NT * n_pad, HB), jnp.int32),
            pltpu.VMEM_SHARED((n_pad, HD), jnp.float32),
            pltpu.SemaphoreType.DMA,
            pltpu.SemaphoreType.DMA,
            pltpu.SemaphoreType.DMA,
            pltpu.SemaphoreType.DMA,
        ],
    )
    def k(xw_hbm, pfx_hbm, out_hbm,
          stg_v, gbuf_v, rows_v, tab_sh, out_sh,
          gsem0, gsem1, ssem0, ssem1):
        cid = lax.axis_index("c")
        sid = lax.axis_index("s")
        gsem = (gsem0, gsem1)
        ssem = (ssem0, ssem1)

        zero16 = jnp.zeros((LANES,), jnp.float32)

        @pl.loop(0, CHUNK)
        def _(r):
            for f in range(HD // LANES):
                rows_v[0, r, pl.ds(f * LANES, LANES)] = zero16

        for i in range(nzero):
            pltpu.sync_copy(
                rows_v.at[0],
                out_sh.at[pl.ds(sid * rows_per_tile + i * CHUNK, CHUNK)])

        pltpu.sync_copy(xw_hbm.at[cid, pl.ds(sid * trows, trows)],
                        tab_sh.at[pl.ds(sid * trows, trows)])

        plsc.subcore_barrier()

        pltpu.sync_copy(pfx_hbm.at[sid, 0], stg_v.at[0])
        for q in (0, 1):
            pltpu.async_copy(tab_sh.at[stg_v.at[0, 0, q]], gbuf_v.at[q],
                             gsem[q])

        mask_hi = jnp.full((LANES,), -65536, jnp.int32)

        @pl.loop(0, nchunk, step=2)
        def _(t):
            for q in (0, 1):
                c = t + q
                ci = lax.rem(c, IB)
                bq = lax.rem(lax.div(c, IB), 2)

                pltpu.make_async_copy(
                    tab_sh.at[stg_v.at[bq, 0, ci]], gbuf_v.at[q],
                    gsem[q]).wait()

                @pl.when(c >= 2)
                def _():
                    pltpu.make_async_copy(
                        rows_v.at[q], out_sh.at[pl.ds(0, CHUNK)],
                        ssem[q]).wait()

                @plsc.parallel_loop(0, CHUNK, step=LANES, unroll=4)
                def _(j):
                    pk16 = plsc.bitcast(stg_v[bq, 2, ci, pl.ds(j, LANES)],
                                        jnp.float32)
                    for l in range(LANES):
                        pkv = jnp.broadcast_to(pk16[l], (LANES,))
                        for g in range(HB // LANES):
                            w16 = gbuf_v[q, j + l, pl.ds(g * LANES, LANES)]
                            flo = plsc.bitcast(w16 << 16, jnp.float32)
                            fhi = plsc.bitcast(w16 & mask_hi, jnp.float32)
                            rows_v[q, j + l, pl.ds(g * LANES, LANES)] = \
                                flo * pkv
                            rows_v[q, j + l,
                                   pl.ds(HB + g * LANES, LANES)] = fhi * pkv

                pltpu.async_copy(rows_v.at[q], out_sh.at[stg_v.at[bq, 1, ci]],
                                 ssem[q], add=True)

                @pl.when(c + 2 < nchunk)
                def _():
                    c2 = c + 2
                    ci2 = lax.rem(c2, IB)
                    blk2 = lax.div(c2, IB)
                    bq2 = lax.rem(blk2, 2)

                    @pl.when(ci2 == 0)
                    def _():
                        pltpu.sync_copy(pfx_hbm.at[sid, blk2], stg_v.at[bq2])

                    pltpu.async_copy(tab_sh.at[stg_v.at[bq2, 0, ci2]],
                                     gbuf_v.at[q], gsem[q])

        for q in (0, 1):
            pltpu.make_async_copy(
                rows_v.at[q], out_sh.at[pl.ds(0, CHUNK)], ssem[q]).wait()

        plsc.subcore_barrier()

        for i in range(nzero):
            rs = sid * rows_per_tile + i * CHUNK
            pltpu.sync_copy(out_sh.at[pl.ds(rs, CHUNK)],
                            out_hbm.at[cid, pl.ds(rs, CHUNK)])

    return k(xw2, pfx)


def _finalize(out_part, den, root, n_pad):
    grid = (n_pad // BN,)

    def body(op_ref, den_ref, root_ref, o_ref):
        op = op_ref[...]
        dsum = jnp.sum(den_ref[...], axis=0) + 1e-16
        agg = jnp.concatenate([op[0], op[1]], axis=-1)
        o_ref[...] = agg / dsum[:, None] + root_ref[...]

    return pl.pallas_call(
        body,
        grid=grid,
        in_specs=[
            pl.BlockSpec((NC, BN, HD), lambda i: (0, i, 0)),
            pl.BlockSpec((NW, BN), lambda i: (0, i)),
            pl.BlockSpec((BN, D), lambda i: (i, 0)),
        ],
        out_specs=pl.BlockSpec((BN, D), lambda i: (i, 0)),
        out_shape=jax.ShapeDtypeStruct((n_pad, D), jnp.float32),
    )(out_part, den, root)


def kernel(x, edge_index, edge_type, weight, att, root_w, root_b):
    n = x.shape[0]
    e = edge_index.shape[1]
    n_pad = _ceil_to(n, BN)
    ept = _ceil_to(e, NS * CHUNK * IB) // NS
    nblk = ept // (CHUNK * IB)
    e_pad = ept * NS

    x_pad = jnp.pad(x, ((0, n_pad - n), (0, 0)))
    src = jnp.pad(edge_index[0].astype(jnp.int32), (0, e_pad - e))
    dst = jnp.pad(edge_index[1].astype(jnp.int32), (0, e_pad - e),
                  constant_values=n_pad - 1)
    typ = jnp.pad(edge_type.astype(jnp.int32), (0, e_pad - e))
    cmb = jnp.stack([src.reshape(NS, nblk, IB, CHUNK),
                     dst.reshape(NS, nblk, IB, CHUNK),
                     typ.reshape(NS, nblk, IB, CHUNK)], axis=2)

    xw, sd, root = _precompute(x_pad, weight, att, root_w,
                               root_b.reshape(1, D), n_pad)
    bf = xw.astype(jnp.bfloat16)
    plo = jax.lax.bitcast_convert_type(bf[..., :HB],
                                       jnp.uint16).astype(jnp.uint32)
    phi = jax.lax.bitcast_convert_type(bf[..., HB:],
                                       jnp.uint16).astype(jnp.uint32)
    xw2 = jax.lax.bitcast_convert_type(plo | (phi << 16), jnp.int32)
    xw2 = xw2.reshape(NC, NT * n_pad, HB)
    s_flat = sd[0:2].reshape(-1)
    d_flat = sd[2:4].reshape(-1)

    pfx, den = _sc_phase1(s_flat, d_flat, cmb, n_pad, nblk)
    out_part = _sc_aggregate(xw2, pfx, n_pad, nblk)
    out = _finalize(out_part, den, root, n_pad)
    return out[:n]


# speedup vs baseline: 1.0442x; 1.0442x over previous
import dataclasses
import functools

import jax
import jax.numpy as jnp
from jax import lax
from jax.experimental import pallas as pl
from jax.experimental.pallas import tpu as pltpu
from jax.experimental.pallas import tpu_sc as plsc

D = 128
NT = 2
NEG = 0.2
NC = 2
NS = 16
NW = NC * NS
LANES = 16
CHUNK = 128
IB = 16
HD = D // NC
HB = HD // 2
BN = 1024


def _ceil_to(v, m):
    return -(-v // m) * m


def _sc_params():
    cp = pltpu.CompilerParams()
    if "needs_layout_passes" in pltpu.CompilerParams.__dataclass_fields__:
        cp = dataclasses.replace(cp, needs_layout_passes=False)
    if "use_tc_tiling_on_sc" in pltpu.CompilerParams.__dataclass_fields__:
        cp = dataclasses.replace(cp, use_tc_tiling_on_sc=False)
    return cp


def _precompute(x_pad, weight, att, root_w, root_b2, n_pad):
    grid = (n_pad // BN,)

    def body(x_ref, w_ref, a_ref, rw_ref, rb_ref, xw_ref, sd_ref, root_ref):
        xb = x_ref[...]
        w = w_ref[...]
        xw0 = jnp.dot(xb, w[0], preferred_element_type=jnp.float32)
        xw1 = jnp.dot(xb, w[1], preferred_element_type=jnp.float32)
        a = a_ref[...]
        s0 = jnp.sum(xw0 * a[0, :D][None, :], axis=1)
        s1 = jnp.sum(xw1 * a[1, :D][None, :], axis=1)
        d0 = jnp.sum(xw0 * a[0, D:][None, :], axis=1)
        d1 = jnp.sum(xw1 * a[1, D:][None, :], axis=1)
        sd_ref[...] = jnp.stack([s0, s1, d0, d1, s0, s1, d0, d1], axis=0)
        lo = jnp.stack([xw0[:, :HD], xw1[:, :HD]])
        hi = jnp.stack([xw0[:, HD:], xw1[:, HD:]])
        xw_ref[...] = jnp.stack([lo, hi])
        root_ref[...] = (jnp.dot(xb, rw_ref[...],
                                 preferred_element_type=jnp.float32)
                         + rb_ref[...])

    return pl.pallas_call(
        body,
        grid=grid,
        in_specs=[
            pl.BlockSpec((BN, D), lambda i: (i, 0)),
            pl.BlockSpec((NT, D, D), lambda i: (0, 0, 0)),
            pl.BlockSpec((NT, 2 * D), lambda i: (0, 0)),
            pl.BlockSpec((D, D), lambda i: (0, 0)),
            pl.BlockSpec((1, D), lambda i: (0, 0)),
        ],
        out_specs=[
            pl.BlockSpec((NC, NT, BN, HD), lambda i: (0, 0, i, 0)),
            pl.BlockSpec((8, BN), lambda i: (0, i)),
            pl.BlockSpec((BN, D), lambda i: (i, 0)),
        ],
        out_shape=[
            jax.ShapeDtypeStruct((NC, NT, n_pad, HD), jnp.float32),
            jax.ShapeDtypeStruct((8, n_pad), jnp.float32),
            jax.ShapeDtypeStruct((n_pad, D), jnp.float32),
        ],
    )(x_pad, weight, att, root_w, root_b2)


def _sc_phase1(s_flat, d_flat, cmb, n_pad, nblk):
    mesh = plsc.VectorSubcoreMesh(core_axis_name="c", subcore_axis_name="s")

    @functools.partial(
        pl.kernel,
        compiler_params=_sc_params(),
        out_type=[
            jax.ShapeDtypeStruct((NS, nblk, 3, IB, CHUNK), jnp.int32),
            jax.ShapeDtypeStruct((NW, n_pad), jnp.float32),
        ],
        mesh=mesh,
        scratch_types=[
            pltpu.VMEM((3, IB, CHUNK), jnp.int32),
            pltpu.VMEM((3, IB, CHUNK), jnp.int32),
            pltpu.VMEM((NT * n_pad,), jnp.float32),
            pltpu.VMEM((NT * n_pad,), jnp.float32),
            pltpu.VMEM((n_pad,), jnp.float32),
        ],
    )
    def k(s_hbm, d_hbm, cmb_hbm, pfx_hbm, den_hbm,
          cin_v, cout_v, s_v, d_v, den_v):
        cid = lax.axis_index("c")
        sid = lax.axis_index("s")

        zero16 = jnp.zeros((LANES,), jnp.float32)

        @pl.loop(0, n_pad, step=LANES)
        def _(i):
            den_v[pl.ds(i, LANES)] = zero16

        pltpu.sync_copy(s_hbm, s_v)
        pltpu.sync_copy(d_hbm, d_v)

        @pl.loop(cid, nblk, step=2)
        def _(blk):
            pltpu.sync_copy(cmb_hbm.at[sid, blk], cin_v)

            @pl.loop(0, IB)
            def _(ci):
                @plsc.parallel_loop(0, CHUNK, step=LANES, unroll=2)
                def _(j):
                    src16 = cin_v[0, ci, pl.ds(j, LANES)]
                    dst16 = cin_v[1, ci, pl.ds(j, LANES)]
                    typ16 = cin_v[2, ci, pl.ds(j, LANES)]
                    fs = typ16 * n_pad + src16
                    fd = typ16 * n_pad + dst16
                    sg = plsc.load_gather(s_v, [fs])
                    dg = plsc.load_gather(d_v, [fd])
                    logit = sg + dg
                    e = jnp.where(logit >= 0, logit, logit * NEG)
                    pe = jnp.exp(e)
                    cout_v[0, ci, pl.ds(j, LANES)] = fs
                    cout_v[1, ci, pl.ds(j, LANES)] = dst16
                    cout_v[2, ci, pl.ds(j, LANES)] = plsc.bitcast(pe,
                                                                  jnp.int32)
                    plsc.addupdate_scatter(den_v, [dst16], pe)

            pltpu.sync_copy(cout_v, pfx_hbm.at[sid, blk])

        pltpu.sync_copy(den_v, den_hbm.at[cid * NS + sid])

    return k(s_flat, d_flat, cmb)


def _sc_aggregate(xw2, pfx, n_pad, nblk):
    mesh = plsc.VectorSubcoreMesh(core_axis_name="c", subcore_axis_name="s")
    rows_per_tile = n_pad // NS
    nzero = rows_per_tile // CHUNK
    trows = NT * n_pad // NS
    nchunk = nblk * IB

    @functools.partial(
        pl.kernel,
        compiler_params=_sc_params(),
        out_type=jax.ShapeDtypeStruct((NC, n_pad, HD), jnp.float32),
        mesh=mesh,
        scratch_types=[
            pltpu.VMEM((2, 3, IB, CHUNK), jnp.int32),
            pltpu.VMEM((2, CHUNK, HB), jnp.int32),
            pltpu.VMEM((2, CHUNK, HD), jnp.float32),
            pltpu.VMEM_SHARED((NT * n_pad, HB), jnp.int32),
            pltpu.VMEM_SHARED((n_pad, HD), jnp.float32),
            pltpu.SemaphoreType.DMA,
            pltpu.SemaphoreType.DMA,
            pltpu.SemaphoreType.DMA,
            pltpu.SemaphoreType.DMA,
        ],
    )
    def k(xw_hbm, pfx_hbm, out_hbm,
          stg_v, gbuf_v, rows_v, tab_sh, out_sh,
          gsem0, gsem1, ssem0, ssem1):
        cid = lax.axis_index("c")
        sid = lax.axis_index("s")
        gsem = (gsem0, gsem1)
        ssem = (ssem0, ssem1)

        zero16 = jnp.zeros((LANES,), jnp.float32)

        @pl.loop(0, CHUNK)
        def _(r):
            for f in range(HD // LANES):
                rows_v[0, r, pl.ds(f * LANES, LANES)] = zero16

        for i in range(nzero):
            pltpu.sync_copy(
                rows_v.at[0],
                out_sh.at[pl.ds(sid * rows_per_tile + i * CHUNK, CHUNK)])

        pltpu.sync_copy(xw_hbm.at[cid, pl.ds(sid * trows, trows)],
                        tab_sh.at[pl.ds(sid * trows, trows)])

        plsc.subcore_barrier()

        pltpu.sync_copy(pfx_hbm.at[sid, 0], stg_v.at[0])
        for q in (0, 1):
            pltpu.async_copy(tab_sh.at[stg_v.at[0, 0, q]], gbuf_v.at[q],
                             gsem[q])

        mask_hi = jnp.full((LANES,), -65536, jnp.int32)

        @pl.loop(0, nchunk, step=2)
        def _(t):
            for q in (0, 1):
                c = t + q
                ci = lax.rem(c, IB)
                bq = lax.rem(lax.div(c, IB), 2)

                pltpu.make_async_copy(
                    tab_sh.at[stg_v.at[bq, 0, ci]], gbuf_v.at[q],
                    gsem[q]).wait()

                @pl.when(c >= 2)
                def _():
                    pltpu.make_async_copy(
                        rows_v.at[q], out_sh.at[pl.ds(0, CHUNK)],
                        ssem[q]).wait()

                @plsc.parallel_loop(0, CHUNK, step=LANES, unroll=2)
                def _(j):
                    pk16 = plsc.bitcast(stg_v[bq, 2, ci, pl.ds(j, LANES)],
                                        jnp.float32)
                    for l in range(LANES):
                        pkv = jnp.broadcast_to(pk16[l], (LANES,))
                        for g in range(HB // LANES):
                            w16 = gbuf_v[q, j + l, pl.ds(g * LANES, LANES)]
                            flo = plsc.bitcast(w16 << 16, jnp.float32)
                            fhi = plsc.bitcast(w16 & mask_hi, jnp.float32)
                            rows_v[q, j + l, pl.ds(g * LANES, LANES)] = \
                                flo * pkv
                            rows_v[q, j + l,
                                   pl.ds(HB + g * LANES, LANES)] = fhi * pkv

                pltpu.async_copy(rows_v.at[q], out_sh.at[stg_v.at[bq, 1, ci]],
                                 ssem[q], add=True)

                @pl.when(c + 2 < nchunk)
                def _():
                    c2 = c + 2
                    ci2 = lax.rem(c2, IB)
                    blk2 = lax.div(c2, IB)
                    bq2 = lax.rem(blk2, 2)

                    @pl.when(ci2 == 0)
                    def _():
                        pltpu.sync_copy(pfx_hbm.at[sid, blk2], stg_v.at[bq2])

                    pltpu.async_copy(tab_sh.at[stg_v.at[bq2, 0, ci2]],
                                     gbuf_v.at[q], gsem[q])

        for q in (0, 1):
            pltpu.make_async_copy(
                rows_v.at[q], out_sh.at[pl.ds(0, CHUNK)], ssem[q]).wait()

        plsc.subcore_barrier()

        for i in range(nzero):
            rs = sid * rows_per_tile + i * CHUNK
            pltpu.sync_copy(out_sh.at[pl.ds(rs, CHUNK)],
                            out_hbm.at[cid, pl.ds(rs, CHUNK)])

    return k(xw2, pfx)


def _finalize(out_part, den, root, n_pad):
    grid = (n_pad // BN,)

    def body(op_ref, den_ref, root_ref, o_ref):
        op = op_ref[...]
        dsum = jnp.sum(den_ref[...], axis=0) + 1e-16
        agg = jnp.concatenate([op[0], op[1]], axis=-1)
        o_ref[...] = agg / dsum[:, None] + root_ref[...]

    return pl.pallas_call(
        body,
        grid=grid,
        in_specs=[
            pl.BlockSpec((NC, BN, HD), lambda i: (0, i, 0)),
            pl.BlockSpec((NW, BN), lambda i: (0, i)),
            pl.BlockSpec((BN, D), lambda i: (i, 0)),
        ],
        out_specs=pl.BlockSpec((BN, D), lambda i: (i, 0)),
        out_shape=jax.ShapeDtypeStruct((n_pad, D), jnp.float32),
    )(out_part, den, root)


def kernel(x, edge_index, edge_type, weight, att, root_w, root_b):
    n = x.shape[0]
    e = edge_index.shape[1]
    n_pad = _ceil_to(n, BN)
    ept = _ceil_to(e, NS * CHUNK * IB) // NS
    nblk = ept // (CHUNK * IB)
    e_pad = ept * NS

    x_pad = jnp.pad(x, ((0, n_pad - n), (0, 0)))
    src = jnp.pad(edge_index[0].astype(jnp.int32), (0, e_pad - e))
    dst = jnp.pad(edge_index[1].astype(jnp.int32), (0, e_pad - e),
                  constant_values=n_pad - 1)
    typ = jnp.pad(edge_type.astype(jnp.int32), (0, e_pad - e))
    cmb = jnp.stack([src.reshape(NS, nblk, IB, CHUNK),
                     dst.reshape(NS, nblk, IB, CHUNK),
                     typ.reshape(NS, nblk, IB, CHUNK)], axis=2)

    xw, sd, root = _precompute(x_pad, weight, att, root_w,
                               root_b.reshape(1, D), n_pad)
    bf = xw.astype(jnp.bfloat16)
    plo = jax.lax.bitcast_convert_type(bf[..., :HB],
                                       jnp.uint16).astype(jnp.uint32)
    phi = jax.lax.bitcast_convert_type(bf[..., HB:],
                                       jnp.uint16).astype(jnp.uint32)
    xw2 = jax.lax.bitcast_convert_type(plo | (phi << 16), jnp.int32)
    xw2 = xw2.reshape(NC, NT * n_pad, HB)
    s_flat = sd[0:2].reshape(-1)
    d_flat = sd[2:4].reshape(-1)

    pfx, den = _sc_phase1(s_flat, d_flat, cmb, n_pad, nblk)
    out_part = _sc_aggregate(xw2, pfx, n_pad, nblk)
    out = _finalize(out_part, den, root, n_pad)
    return out[:n]


# speedup vs baseline: 1.0454x; 1.0011x over previous
import dataclasses
import functools

import jax
import jax.numpy as jnp
from jax import lax
from jax.experimental import pallas as pl
from jax.experimental.pallas import tpu as pltpu
from jax.experimental.pallas import tpu_sc as plsc

D = 128
NT = 2
NEG = 0.2
NC = 2
NS = 16
NW = NC * NS
LANES = 16
CHUNK = 128
IB = 16
HD = D // NC
HB = HD // 2
BN = 1024


def _ceil_to(v, m):
    return -(-v // m) * m


def _sc_params():
    cp = pltpu.CompilerParams()
    if "needs_layout_passes" in pltpu.CompilerParams.__dataclass_fields__:
        cp = dataclasses.replace(cp, needs_layout_passes=False)
    if "use_tc_tiling_on_sc" in pltpu.CompilerParams.__dataclass_fields__:
        cp = dataclasses.replace(cp, use_tc_tiling_on_sc=False)
    return cp


def _precompute(x_pad, weight, att, root_w, root_b2, n_pad):
    grid = (n_pad // BN,)

    def body(x_ref, w_ref, a_ref, rw_ref, rb_ref, xw_ref, sd_ref, root_ref):
        xb = x_ref[...]
        w = w_ref[...]
        xw0 = jnp.dot(xb, w[0], preferred_element_type=jnp.float32)
        xw1 = jnp.dot(xb, w[1], preferred_element_type=jnp.float32)
        a = a_ref[...]
        s0 = jnp.sum(xw0 * a[0, :D][None, :], axis=1)
        s1 = jnp.sum(xw1 * a[1, :D][None, :], axis=1)
        d0 = jnp.sum(xw0 * a[0, D:][None, :], axis=1)
        d1 = jnp.sum(xw1 * a[1, D:][None, :], axis=1)
        sd_ref[...] = jnp.stack([s0, s1, d0, d1, s0, s1, d0, d1], axis=0)
        lo = jnp.stack([xw0[:, :HD], xw1[:, :HD]])
        hi = jnp.stack([xw0[:, HD:], xw1[:, HD:]])
        xw_ref[...] = jnp.stack([lo, hi])
        root_ref[...] = (jnp.dot(xb, rw_ref[...],
                                 preferred_element_type=jnp.float32)
                         + rb_ref[...])

    return pl.pallas_call(
        body,
        grid=grid,
        in_specs=[
            pl.BlockSpec((BN, D), lambda i: (i, 0)),
            pl.BlockSpec((NT, D, D), lambda i: (0, 0, 0)),
            pl.BlockSpec((NT, 2 * D), lambda i: (0, 0)),
            pl.BlockSpec((D, D), lambda i: (0, 0)),
            pl.BlockSpec((1, D), lambda i: (0, 0)),
        ],
        out_specs=[
            pl.BlockSpec((NC, NT, BN, HD), lambda i: (0, 0, i, 0)),
            pl.BlockSpec((8, BN), lambda i: (0, i)),
            pl.BlockSpec((BN, D), lambda i: (i, 0)),
        ],
        out_shape=[
            jax.ShapeDtypeStruct((NC, NT, n_pad, HD), jnp.float32),
            jax.ShapeDtypeStruct((8, n_pad), jnp.float32),
            jax.ShapeDtypeStruct((n_pad, D), jnp.float32),
        ],
    )(x_pad, weight, att, root_w, root_b2)


def _sc_phase1(s_flat, d_flat, cmb, n_pad, nblk):
    mesh = plsc.VectorSubcoreMesh(core_axis_name="c", subcore_axis_name="s")

    @functools.partial(
        pl.kernel,
        compiler_params=_sc_params(),
        out_type=[
            jax.ShapeDtypeStruct((NS, nblk, 3, IB, CHUNK), jnp.int32),
            jax.ShapeDtypeStruct((NW, n_pad), jnp.float32),
        ],
        mesh=mesh,
        scratch_types=[
            pltpu.VMEM((2, 3, IB, CHUNK), jnp.int32),
            pltpu.VMEM((2, 3, IB, CHUNK), jnp.int32),
            pltpu.VMEM((NT * n_pad,), jnp.float32),
            pltpu.VMEM((NT * n_pad,), jnp.float32),
            pltpu.VMEM((n_pad,), jnp.float32),
            pltpu.SemaphoreType.DMA,
            pltpu.SemaphoreType.DMA,
            pltpu.SemaphoreType.DMA,
            pltpu.SemaphoreType.DMA,
        ],
    )
    def k(s_hbm, d_hbm, cmb_hbm, pfx_hbm, den_hbm,
          cin_v, cout_v, s_v, d_v, den_v, isem0, isem1, osem0, osem1):
        cid = lax.axis_index("c")
        sid = lax.axis_index("s")
        isem = (isem0, isem1)
        osem = (osem0, osem1)

        zero16 = jnp.zeros((LANES,), jnp.float32)

        pltpu.async_copy(cmb_hbm.at[sid, cid], cin_v.at[0], isem0)

        @pl.loop(0, n_pad, step=LANES)
        def _(i):
            den_v[pl.ds(i, LANES)] = zero16

        pltpu.sync_copy(s_hbm, s_v)
        pltpu.sync_copy(d_hbm, d_v)

        nb2 = nblk // 2
        for i in range(nb2):
            u = i % 2
            blk = cid + 2 * i
            pltpu.make_async_copy(cmb_hbm.at[sid, blk], cin_v.at[u],
                                  isem[u]).wait()
            if i + 1 < nb2:
                pltpu.async_copy(cmb_hbm.at[sid, blk + 2], cin_v.at[1 - u],
                                 isem[1 - u])
            if i >= 2:
                pltpu.make_async_copy(cout_v.at[u], pfx_hbm.at[sid, blk],
                                      osem[u]).wait()

            @pl.loop(0, IB)
            def _(ci):
                @plsc.parallel_loop(0, CHUNK, step=LANES, unroll=2)
                def _(j):
                    src16 = cin_v[u, 0, ci, pl.ds(j, LANES)]
                    dst16 = cin_v[u, 1, ci, pl.ds(j, LANES)]
                    typ16 = cin_v[u, 2, ci, pl.ds(j, LANES)]
                    fs = typ16 * n_pad + src16
                    fd = typ16 * n_pad + dst16
                    sg = plsc.load_gather(s_v, [fs])
                    dg = plsc.load_gather(d_v, [fd])
                    logit = sg + dg
                    e = jnp.where(logit >= 0, logit, logit * NEG)
                    pe = jnp.exp(e)
                    cout_v[u, 0, ci, pl.ds(j, LANES)] = fs
                    cout_v[u, 1, ci, pl.ds(j, LANES)] = dst16
                    cout_v[u, 2, ci, pl.ds(j, LANES)] = plsc.bitcast(
                        pe, jnp.int32)
                    plsc.addupdate_scatter(den_v, [dst16], pe)

            pltpu.async_copy(cout_v.at[u], pfx_hbm.at[sid, blk], osem[u])

        for i in range(max(nb2 - 2, 0), nb2):
            u = i % 2
            pltpu.make_async_copy(cout_v.at[u],
                                  pfx_hbm.at[sid, cid + 2 * i],
                                  osem[u]).wait()

        pltpu.sync_copy(den_v, den_hbm.at[cid * NS + sid])

    return k(s_flat, d_flat, cmb)


def _sc_aggregate(xw2, pfx, n_pad, nblk):
    mesh = plsc.VectorSubcoreMesh(core_axis_name="c", subcore_axis_name="s")
    rows_per_tile = n_pad // NS
    nzero = rows_per_tile // CHUNK
    trows = NT * n_pad // NS
    nchunk = nblk * IB

    @functools.partial(
        pl.kernel,
        compiler_params=_sc_params(),
        out_type=jax.ShapeDtypeStruct((NC, n_pad, HD), jnp.float32),
        mesh=mesh,
        scratch_types=[
            pltpu.VMEM((2, 3, IB, CHUNK), jnp.int32),
            pltpu.VMEM((2, CHUNK, HB), jnp.int32),
            pltpu.VMEM((2, CHUNK, HD), jnp.float32),
            pltpu.VMEM_SHARED((NT * n_pad, HB), jnp.int32),
            pltpu.VMEM_SHARED((n_pad, HD), jnp.float32),
            pltpu.SemaphoreType.DMA,
            pltpu.SemaphoreType.DMA,
            pltpu.SemaphoreType.DMA,
            pltpu.SemaphoreType.DMA,
        ],
    )
    def k(xw_hbm, pfx_hbm, out_hbm,
          stg_v, gbuf_v, rows_v, tab_sh, out_sh,
          gsem0, gsem1, ssem0, ssem1):
        cid = lax.axis_index("c")
        sid = lax.axis_index("s")
        gsem = (gsem0, gsem1)
        ssem = (ssem0, ssem1)

        zero16 = jnp.zeros((LANES,), jnp.float32)

        @pl.loop(0, CHUNK)
        def _(r):
            for f in range(HD // LANES):
                rows_v[0, r, pl.ds(f * LANES, LANES)] = zero16

        for i in range(nzero):
            pltpu.sync_copy(
                rows_v.at[0],
                out_sh.at[pl.ds(sid * rows_per_tile + i * CHUNK, CHUNK)])

        pltpu.sync_copy(xw_hbm.at[cid, pl.ds(sid * trows, trows)],
                        tab_sh.at[pl.ds(sid * trows, trows)])

        plsc.subcore_barrier()

        pltpu.sync_copy(pfx_hbm.at[sid, 0], stg_v.at[0])
        for q in (0, 1):
            pltpu.async_copy(tab_sh.at[stg_v.at[0, 0, q]], gbuf_v.at[q],
                             gsem[q])

        mask_hi = jnp.full((LANES,), -65536, jnp.int32)

        @pl.loop(0, nchunk, step=2)
        def _(t):
            for q in (0, 1):
                c = t + q
                ci = lax.rem(c, IB)
                bq = lax.rem(lax.div(c, IB), 2)

                pltpu.make_async_copy(
                    tab_sh.at[stg_v.at[bq, 0, ci]], gbuf_v.at[q],
                    gsem[q]).wait()

                @pl.when(c >= 2)
                def _():
                    pltpu.make_async_copy(
                        rows_v.at[q], out_sh.at[pl.ds(0, CHUNK)],
                        ssem[q]).wait()

                @plsc.parallel_loop(0, CHUNK, step=LANES, unroll=2)
                def _(j):
                    pk16 = plsc.bitcast(stg_v[bq, 2, ci, pl.ds(j, LANES)],
                                        jnp.float32)
                    for l in range(LANES):
                        pkv = jnp.broadcast_to(pk16[l], (LANES,))
                        for g in range(HB // LANES):
                            w16 = gbuf_v[q, j + l, pl.ds(g * LANES, LANES)]
                            flo = plsc.bitcast(w16 << 16, jnp.float32)
                            fhi = plsc.bitcast(w16 & mask_hi, jnp.float32)
                            rows_v[q, j + l, pl.ds(g * LANES, LANES)] = \
                                flo * pkv
                            rows_v[q, j + l,
                                   pl.ds(HB + g * LANES, LANES)] = fhi * pkv

                pltpu.async_copy(rows_v.at[q], out_sh.at[stg_v.at[bq, 1, ci]],
                                 ssem[q], add=True)

                @pl.when(c + 2 < nchunk)
                def _():
                    c2 = c + 2
                    ci2 = lax.rem(c2, IB)
                    blk2 = lax.div(c2, IB)
                    bq2 = lax.rem(blk2, 2)

                    @pl.when(ci2 == 0)
                    def _():
                        pltpu.sync_copy(pfx_hbm.at[sid, blk2], stg_v.at[bq2])

                    pltpu.async_copy(tab_sh.at[stg_v.at[bq2, 0, ci2]],
                                     gbuf_v.at[q], gsem[q])

        for q in (0, 1):
            pltpu.make_async_copy(
                rows_v.at[q], out_sh.at[pl.ds(0, CHUNK)], ssem[q]).wait()

        plsc.subcore_barrier()

        for i in range(nzero):
            rs = sid * rows_per_tile + i * CHUNK
            pltpu.sync_copy(out_sh.at[pl.ds(rs, CHUNK)],
                            out_hbm.at[cid, pl.ds(rs, CHUNK)])

    return k(xw2, pfx)


def _finalize(out_part, den, root, n_pad):
    grid = (n_pad // BN,)

    def body(op_ref, den_ref, root_ref, o_ref):
        op = op_ref[...]
        dsum = jnp.sum(den_ref[...], axis=0) + 1e-16
        agg = jnp.concatenate([op[0], op[1]], axis=-1)
        o_ref[...] = agg / dsum[:, None] + root_ref[...]

    return pl.pallas_call(
        body,
        grid=grid,
        in_specs=[
            pl.BlockSpec((NC, BN, HD), lambda i: (0, i, 0)),
            pl.BlockSpec((NW, BN), lambda i: (0, i)),
            pl.BlockSpec((BN, D), lambda i: (i, 0)),
        ],
        out_specs=pl.BlockSpec((BN, D), lambda i: (i, 0)),
        out_shape=jax.ShapeDtypeStruct((n_pad, D), jnp.float32),
    )(out_part, den, root)


def kernel(x, edge_index, edge_type, weight, att, root_w, root_b):
    n = x.shape[0]
    e = edge_index.shape[1]
    n_pad = _ceil_to(n, BN)
    ept = _ceil_to(e, NS * CHUNK * IB * 2) // NS
    nblk = ept // (CHUNK * IB)
    e_pad = ept * NS

    x_pad = jnp.pad(x, ((0, n_pad - n), (0, 0)))
    src = jnp.pad(edge_index[0].astype(jnp.int32), (0, e_pad - e))
    dst = jnp.pad(edge_index[1].astype(jnp.int32), (0, e_pad - e),
                  constant_values=n_pad - 1)
    typ = jnp.pad(edge_type.astype(jnp.int32), (0, e_pad - e))
    cmb = jnp.stack([src.reshape(NS, nblk, IB, CHUNK),
                     dst.reshape(NS, nblk, IB, CHUNK),
                     typ.reshape(NS, nblk, IB, CHUNK)], axis=2)

    xw, sd, root = _precompute(x_pad, weight, att, root_w,
                               root_b.reshape(1, D), n_pad)
    bf = xw.astype(jnp.bfloat16)
    plo = jax.lax.bitcast_convert_type(bf[..., :HB],
                                       jnp.uint16).astype(jnp.uint32)
    phi = jax.lax.bitcast_convert_type(bf[..., HB:],
                                       jnp.uint16).astype(jnp.uint32)
    xw2 = jax.lax.bitcast_convert_type(plo | (phi << 16), jnp.int32)
    xw2 = xw2.reshape(NC, NT * n_pad, HB)
    s_flat = sd[0:2].reshape(-1)
    d_flat = sd[2:4].reshape(-1)

    pfx, den = _sc_phase1(s_flat, d_flat, cmb, n_pad, nblk)
    out_part = _sc_aggregate(xw2, pfx, n_pad, nblk)
    out = _finalize(out_part, den, root, n_pad)
    return out[:n]
